# baseline (speedup 1.0000x reference)
import functools

import jax
import jax.numpy as jnp
from jax import lax
from jax.experimental import pallas as pl
from jax.experimental.pallas import tpu as pltpu, tpu_sc as plsc

MC = 10000
MCP = 10240
NE = 320000
H = 128
ETN = 4
FIN = 148
ITERS = 2

NC = 2
NS = 16
NW = NC * NS
EPW = NE // NW
K = 40
NCHUNK = EPW // K
RING = 3
RINGI = 4
UNROLL = 12
NSTEP = NCHUNK + 2
NGROUP = NSTEP // UNROLL
RPT = MCP // NS
ZR = K

_F32 = jnp.float32


def _lrelu(x):
    return jnp.maximum(x, x * 0.01)


def _prep_body(child_ref, geo_ref, ex_ref, wc_ref, bc_ref, wne_ref, bne_ref,
               wg_ref, bg_ref, wsk_ref, bsk_ref,
               a_ref, c_ref, sums_ref, acc, bmat_s):
    i = pl.program_id(0)
    t = pl.program_id(1)
    n = pl.num_programs(0)

    @pl.when(jnp.logical_and(i == 0, t == 0))
    def _():
        acc[...] = jnp.zeros_like(acc)

    @pl.when(t == 0)
    def _():
        ex = ex_ref[...]
        cf = jnp.dot(child_ref[...], wc_ref[...], preferred_element_type=_F32)
        cf = (cf + bc_ref[...]) * ex
        a_ref[...] = jnp.dot(cf, wne_ref[0:H, :], preferred_element_type=_F32)
        bmat_s[...] = jnp.dot(cf, wne_ref[H:2 * H, :],
                              preferred_element_type=_F32)

        g = jnp.dot(geo_ref[...], wg_ref[...], preferred_element_type=_F32)
        g = (g + bg_ref[...]) * ex
        sk = jnp.dot(geo_ref[...], wsk_ref[...], preferred_element_type=_F32)
        sk = (sk + bsk_ref[...]) * ex
        acc[0:1, :] += jnp.sum(cf, axis=0, keepdims=True)
        acc[1:2, :] += jnp.sum(g, axis=0, keepdims=True)
        acc[2:3, :] += jnp.sum(sk, axis=0, keepdims=True)

    tb = wne_ref[2 * H:2 * H + ETN, :] + bne_ref[...]
    onehot = (lax.broadcasted_iota(jnp.int32, (1, ETN), 1) == t).astype(_F32)
    tbrow = jnp.dot(onehot, tb, preferred_element_type=_F32)
    c_ref[...] = bmat_s[...] + tbrow

    @pl.when(jnp.logical_and(i == n - 1, t == ETN - 1))
    def _():
        sums_ref[...] = acc[...]


def _tc_prep(child, geo, ex, wc, bc, wne, bne, wg, bg, wsk, bsk, r=1000):
    grid = (MC // r, ETN)
    return pl.pallas_call(
        _prep_body,
        grid=grid,
        in_specs=[
            pl.BlockSpec((r, FIN), lambda i, t: (i, 0)),
            pl.BlockSpec((r, H), lambda i, t: (i, 0)),
            pl.BlockSpec((r, 1), lambda i, t: (i, 0)),
            pl.BlockSpec((FIN, H), lambda i, t: (0, 0)),
            pl.BlockSpec((1, H), lambda i, t: (0, 0)),
            pl.BlockSpec((2 * H + ETN, H), lambda i, t: (0, 0)),
            pl.BlockSpec((1, H), lambda i, t: (0, 0)),
            pl.BlockSpec((H, H), lambda i, t: (0, 0)),
            pl.BlockSpec((1, H), lambda i, t: (0, 0)),
            pl.BlockSpec((H, H), lambda i, t: (0, 0)),
            pl.BlockSpec((1, H), lambda i, t: (0, 0)),
        ],
        out_specs=[
            pl.BlockSpec((r, H), lambda i, t: (i, 0)),
            pl.BlockSpec((r, H), lambda i, t: (t * (MC // r) + i, 0)),
            pl.BlockSpec((8, H), lambda i, t: (0, 0)),
        ],
        out_shape=[
            jax.ShapeDtypeStruct((MC, H), _F32),
            jax.ShapeDtypeStruct((ETN * MC, H), _F32),
            jax.ShapeDtypeStruct((8, H), _F32),
        ],
        scratch_shapes=[pltpu.VMEM((8, H), _F32), pltpu.VMEM((r, H), _F32)],
    )(child, geo, ex, wc, bc, wne, bne, wg, bg, wsk, bsk)


def _mid_body(p_ref, wne_ref, bne_ref, a_ref, c_ref, sums_ref, acc, bmat_s):
    i = pl.program_id(0)
    t = pl.program_id(1)
    n = pl.num_programs(0)

    @pl.when(jnp.logical_and(i == 0, t == 0))
    def _():
        acc[...] = jnp.zeros_like(acc)

    @pl.when(t == 0)
    def _():
        cf = p_ref[0] + p_ref[1]
        a_ref[...] = jnp.dot(cf, wne_ref[0:H, :], preferred_element_type=_F32)
        bmat_s[...] = jnp.dot(cf, wne_ref[H:2 * H, :],
                              preferred_element_type=_F32)
        acc[0:1, :] += jnp.sum(cf, axis=0, keepdims=True)

    tb = wne_ref[2 * H:2 * H + ETN, :] + bne_ref[...]
    onehot = (lax.broadcasted_iota(jnp.int32, (1, ETN), 1) == t).astype(_F32)
    c_ref[...] = bmat_s[...] + jnp.dot(onehot, tb, preferred_element_type=_F32)

    @pl.when(jnp.logical_and(i == n - 1, t == ETN - 1))
    def _():
        sums_ref[...] = acc[...]


def _tc_mid(p, wne, bne, r=1024):
    grid = (MCP // r, ETN)
    return pl.pallas_call(
        _mid_body,
        grid=grid,
        in_specs=[
            pl.BlockSpec((2, r, H), lambda i, t: (0, i, 0)),
            pl.BlockSpec((2 * H + ETN, H), lambda i, t: (0, 0)),
            pl.BlockSpec((1, H), lambda i, t: (0, 0)),
        ],
        out_specs=[
            pl.BlockSpec((r, H), lambda i, t: (i, 0)),
            pl.BlockSpec((r, H), lambda i, t: (t * (MCP // r) + i, 0)),
            pl.BlockSpec((8, H), lambda i, t: (0, 0)),
        ],
        out_shape=[
            jax.ShapeDtypeStruct((MCP, H), _F32),
            jax.ShapeDtypeStruct((ETN * MCP, H), _F32),
            jax.ShapeDtypeStruct((8, H), _F32),
        ],
        scratch_shapes=[pltpu.VMEM((8, H), _F32), pltpu.VMEM((r, H), _F32)],
    )(p, wne, bne)


def _final_body(p_ref, sums1_ref, sums2_ref, ws_ref, bs_ref, wsg_ref, bsg_ref,
                gnw_ref, gnb_ref, pf_ref, pgf_ref, acc):
    i = pl.program_id(0)
    n = pl.num_programs(0)

    @pl.when(i == 0)
    def _():
        acc[...] = jnp.zeros_like(acc)

    cf = p_ref[0] + p_ref[1]
    acc[0:1, :] += jnp.sum(cf, axis=0, keepdims=True)

    @pl.when(i == n - 1)
    def _():
        s0 = sums1_ref[0:1, :]
        gsum = sums1_ref[1:2, :]
        ssum = sums1_ref[2:3, :]
        s1 = sums2_ref[0:1, :]
        s2 = acc[0:1, :]
        pre = (jnp.dot(s0, ws_ref[0:H, :], preferred_element_type=_F32)
               + jnp.dot(s1, ws_ref[H:2 * H, :], preferred_element_type=_F32)
               + jnp.dot(s2, ws_ref[2 * H:3 * H, :], preferred_element_type=_F32)
               + bs_ref[...])
        pf_ref[...] = _lrelu(pre)

        pg = _lrelu(gsum)
        sg = _lrelu(ssum)
        y = jnp.dot(pg, wsg_ref[...], preferred_element_type=_F32) + bsg_ref[...]
        r8 = lax.broadcasted_iota(jnp.int32, (H, H), 0) // 8
        c8 = lax.broadcasted_iota(jnp.int32, (H, H), 1) // 8
        m8 = jnp.where(r8 == c8, 1.0 / 8.0, 0.0).astype(_F32)
        m = jnp.dot(y, m8, preferred_element_type=_F32)
        ex2 = jnp.dot(y * y, m8, preferred_element_type=_F32)
        var = ex2 - m * m
        gn = (y - m) * lax.rsqrt(var + 1e-5) * gnw_ref[...] + gnb_ref[...]
        pgf_ref[...] = _lrelu(sg + gn)


def _tc_final(p, sums1, sums2, ws, bs, wsg, bsg, gnw, gnb, r=1024):
    grid = (MCP // r,)
    return pl.pallas_call(
        _final_body,
        grid=grid,
        in_specs=[
            pl.BlockSpec((2, r, H), lambda i: (0, i, 0)),
            pl.BlockSpec((8, H), lambda i: (0, 0)),
            pl.BlockSpec((8, H), lambda i: (0, 0)),
            pl.BlockSpec((3 * H, H), lambda i: (0, 0)),
            pl.BlockSpec((1, H), lambda i: (0, 0)),
            pl.BlockSpec((H, H), lambda i: (0, 0)),
            pl.BlockSpec((1, H), lambda i: (0, 0)),
            pl.BlockSpec((1, H), lambda i: (0, 0)),
            pl.BlockSpec((1, H), lambda i: (0, 0)),
        ],
        out_specs=[
            pl.BlockSpec((1, H), lambda i: (0, 0)),
            pl.BlockSpec((1, H), lambda i: (0, 0)),
        ],
        out_shape=[
            jax.ShapeDtypeStruct((1, H), _F32),
            jax.ShapeDtypeStruct((1, H), _F32),
        ],
        scratch_shapes=[pltpu.VMEM((8, H), _F32)],
    )(p, sums1, sums2, ws, bs, wsg, bsg, gnw, gnb)


_sc_mesh = plsc.VectorSubcoreMesh(core_axis_name="c", subcore_axis_name="s")


@functools.partial(
    pl.kernel,
    mesh=_sc_mesh,
    out_type=jax.ShapeDtypeStruct((NC, MCP, H), _F32),
    scratch_types=(
        [pltpu.VMEM((2, K), jnp.int32) for _ in range(RINGI)]
        + [pltpu.VMEM((K, H), _F32) for _ in range(2 * RING)]
        + [pltpu.SemaphoreType.DMA for _ in range(RINGI + 2 * RING)]
        + [pltpu.VMEM_SHARED((MCP, H), _F32)]
    ),
)
def _sc_edge_pass(a_hbm, c_hbm, idx_hbm, out_hbm,
                  ix0, ix1, ix2, ix3, ra0, ra1, ra2, rc0, rc1, rc2,
                  si0, si1, si2, si3, sg0, sg1, sg2, ss0, ss1, ss2, acc_sh):
    cid = lax.axis_index("c")
    sid = lax.axis_index("s")
    wid = sid * NC + cid
    ix = (ix0, ix1, ix2, ix3)
    ra = (ra0, ra1, ra2)
    rc = (rc0, rc1, rc2)
    si = (si0, si1, si2, si3)
    sg = (sg0, sg1, sg2)
    ss = (ss0, ss1, ss2)

    zv = jnp.zeros((16,), _F32)

    def _zrow(r, carry):
        for c in range(H // 16):
            ra0[r, pl.ds(c * 16, 16)] = zv
        return carry

    lax.fori_loop(0, K, _zrow, 0)
    for z in range(RPT // ZR):
        pltpu.sync_copy(ra0, acc_sh.at[pl.ds(sid * RPT + z * ZR, ZR)])
    plsc.subcore_barrier()

    def _issue_idx(chunk, bi):
        pltpu.async_copy(idx_hbm.at[wid, chunk], ix[bi], si[bi])

    def _wait_idx(chunk, bi):
        pltpu.make_async_copy(idx_hbm.at[wid, chunk], ix[bi], si[bi]).wait()

    def _issue_gather(bi, b):
        pltpu.async_copy(a_hbm.at[ix[bi].at[0]], ra[b], sg[b])
        pltpu.async_copy(c_hbm.at[ix[bi].at[1]], rc[b], sg[b])

    def _wait_gather(bi, b):
        pltpu.make_async_copy(a_hbm.at[ix[bi].at[0]], ra[b], sg[b]).wait()
        pltpu.make_async_copy(c_hbm.at[ix[bi].at[1]], rc[b], sg[b]).wait()

    _issue_idx(0, 0)
    _issue_idx(1, 1)
    _issue_idx(2, 2)
    _wait_idx(0, 0)
    _issue_gather(0, 0)
    _wait_idx(1, 1)
    _issue_gather(1, 1)

    def _group(g, carry):
        for u in range(UNROLL):
            s = g * UNROLL + u
            b = u % RING
            bp = (u + 2) % RING
            bi = u % RINGI
            bip = (u + 2) % RINGI
            bin_ = (u + 3) % RINGI

            @pl.when(jnp.logical_and(s >= 1, s - 1 < NCHUNK))
            def _():
                pltpu.make_async_copy(
                    ra[bp], acc_sh.at[ix[(u + 3) % RINGI].at[0]],
                    ss[bp]).wait()

            @pl.when(s + 3 < NCHUNK)
            def _():
                _issue_idx(s + 3, bin_)

            @pl.when(s + 2 < NCHUNK)
            def _():
                _wait_idx(s + 2, bip)
                _issue_gather(bip, bp)

            @pl.when(s < NCHUNK)
            def _():
                _wait_gather(bi, b)

                def _row(r, rcarry):
                    for c in range(H // 16):
                        x = (ra[b][r, pl.ds(c * 16, 16)]
                             + rc[b][r, pl.ds(c * 16, 16)])
                        ra[b][r, pl.ds(c * 16, 16)] = jnp.maximum(x, x * 0.01)
                    return rcarry

                lax.fori_loop(0, K, _row, 0)
                pltpu.async_copy(ra[b], acc_sh.at[ix[bi].at[0]], ss[b],
                                 add=True)
        return carry

    lax.fori_loop(0, NGROUP, _group, 0)
    plsc.subcore_barrier()

    for z in range(RPT // ZR):
        r0 = sid * RPT + z * ZR
        pltpu.sync_copy(acc_sh.at[pl.ds(r0, ZR)], ra0)
        pltpu.sync_copy(ra0, out_hbm.at[cid, pl.ds(r0, ZR)])


def kernel(child_feats, child_geo_feats, child_exists, edge_type_onehot,
           edge_indices, W_child, b_child, W_second, b_second, W_ne0, b_ne0,
           W_ne1, b_ne1, W_child_geo, b_child_geo, W_second_geo, b_second_geo,
           gn_w, gn_b, W_skip_geo, b_skip_geo):
    child = child_feats[0]
    geo = child_geo_feats[0]
    ex = child_exists[0]
    e_from = edge_indices[0, :, 0].astype(jnp.int32)
    e_to = edge_indices[0, :, 1].astype(jnp.int32)
    t = jnp.argmax(edge_type_onehot[0], axis=1).astype(jnp.int32)
    eidx2a = t * MC + e_to
    eidx2b = t * MCP + e_to
    e_from_w = e_from.reshape(NW, NCHUNK, K)
    idx_a = jnp.stack([e_from_w, eidx2a.reshape(NW, NCHUNK, K)], axis=2)
    idx_b = jnp.stack([e_from_w, eidx2b.reshape(NW, NCHUNK, K)], axis=2)

    a1, c1, sums1 = _tc_prep(
        child, geo, ex, W_child, b_child[None], W_ne0, b_ne0[None],
        W_child_geo, b_child_geo[None], W_skip_geo, b_skip_geo[None])
    p1 = _sc_edge_pass(a1, c1, idx_a)
    a2, c2, sums2 = _tc_mid(p1, W_ne1, b_ne1[None])
    p2 = _sc_edge_pass(a2, c2, idx_b)
    pf, pgf = _tc_final(p2, sums1, sums2, W_second, b_second[None],
                        W_second_geo, b_second_geo[None],
                        gn_w[None], gn_b[None])
    return pf, pgf


# speedup vs baseline: 1.0803x; 1.0803x over previous
import functools

import jax
import jax.numpy as jnp
from jax import lax
from jax.experimental import pallas as pl
from jax.experimental.pallas import tpu as pltpu, tpu_sc as plsc

MC = 10000
MCP = 10240
NE = 320000
H = 128
ETN = 4
FIN = 148
ITERS = 2

NC = 2
NS = 16
NW = NC * NS
EPW = NE // NW
K = 40
NCHUNK = EPW // K
RING = 3
RINGI = 4
UNROLL = 12
NSTEP = NCHUNK + 2
NGROUP = NSTEP // UNROLL
RPT = MCP // NS
ZR = K

_F32 = jnp.float32


def _lrelu(x):
    return jnp.maximum(x, x * 0.01)


def _prep_body(child_ref, geo_ref, ex_ref, wc_ref, bc_ref, wne_ref, bne_ref,
               wg_ref, bg_ref, wsk_ref, bsk_ref,
               a_ref, c_ref, sums_ref, acc):
    i = pl.program_id(0)
    n = pl.num_programs(0)

    @pl.when(i == 0)
    def _():
        acc[...] = jnp.zeros_like(acc)

    ex = ex_ref[...]
    cf = jnp.dot(child_ref[...], wc_ref[...], preferred_element_type=_F32)
    cf = (cf + bc_ref[...]) * ex
    a_ref[...] = jnp.dot(cf, wne_ref[0:H, :], preferred_element_type=_F32)
    bmat = jnp.dot(cf, wne_ref[H:2 * H, :], preferred_element_type=_F32)
    tb = wne_ref[2 * H:2 * H + ETN, :] + bne_ref[...]
    c_ref[...] = bmat[None] + tb[:, None, :]

    g = jnp.dot(geo_ref[...], wg_ref[...], preferred_element_type=_F32)
    g = (g + bg_ref[...]) * ex
    sk = jnp.dot(geo_ref[...], wsk_ref[...], preferred_element_type=_F32)
    sk = (sk + bsk_ref[...]) * ex

    acc[0:1, :] += jnp.sum(cf, axis=0, keepdims=True)
    acc[1:2, :] += jnp.sum(g, axis=0, keepdims=True)
    acc[2:3, :] += jnp.sum(sk, axis=0, keepdims=True)

    @pl.when(i == n - 1)
    def _():
        sums_ref[...] = acc[...]


def _tc_prep(child, geo, ex, wc, bc, wne, bne, wg, bg, wsk, bsk, r=1000):
    grid = (MC // r,)
    return pl.pallas_call(
        _prep_body,
        grid=grid,
        in_specs=[
            pl.BlockSpec((r, FIN), lambda i: (i, 0)),
            pl.BlockSpec((r, H), lambda i: (i, 0)),
            pl.BlockSpec((r, 1), lambda i: (i, 0)),
            pl.BlockSpec((FIN, H), lambda i: (0, 0)),
            pl.BlockSpec((1, H), lambda i: (0, 0)),
            pl.BlockSpec((2 * H + ETN, H), lambda i: (0, 0)),
            pl.BlockSpec((1, H), lambda i: (0, 0)),
            pl.BlockSpec((H, H), lambda i: (0, 0)),
            pl.BlockSpec((1, H), lambda i: (0, 0)),
            pl.BlockSpec((H, H), lambda i: (0, 0)),
            pl.BlockSpec((1, H), lambda i: (0, 0)),
        ],
        out_specs=[
            pl.BlockSpec((r, H), lambda i: (i, 0)),
            pl.BlockSpec((ETN, r, H), lambda i: (0, i, 0)),
            pl.BlockSpec((8, H), lambda i: (0, 0)),
        ],
        out_shape=[
            jax.ShapeDtypeStruct((MC, H), _F32),
            jax.ShapeDtypeStruct((ETN, MC, H), _F32),
            jax.ShapeDtypeStruct((8, H), _F32),
        ],
        scratch_shapes=[pltpu.VMEM((8, H), _F32)],
    )(child, geo, ex, wc, bc, wne, bne, wg, bg, wsk, bsk)


def _mid_body(p_ref, wne_ref, bne_ref, a_ref, c_ref, sums_ref, acc):
    i = pl.program_id(0)
    n = pl.num_programs(0)

    @pl.when(i == 0)
    def _():
        acc[...] = jnp.zeros_like(acc)

    cf = p_ref[0] + p_ref[1]
    a_ref[...] = jnp.dot(cf, wne_ref[0:H, :], preferred_element_type=_F32)
    bmat = jnp.dot(cf, wne_ref[H:2 * H, :], preferred_element_type=_F32)
    tb = wne_ref[2 * H:2 * H + ETN, :] + bne_ref[...]
    c_ref[...] = bmat[None] + tb[:, None, :]
    acc[0:1, :] += jnp.sum(cf, axis=0, keepdims=True)

    @pl.when(i == n - 1)
    def _():
        sums_ref[...] = acc[...]


def _tc_mid(p, wne, bne, r=1024):
    grid = (MCP // r,)
    return pl.pallas_call(
        _mid_body,
        grid=grid,
        in_specs=[
            pl.BlockSpec((2, r, H), lambda i: (0, i, 0)),
            pl.BlockSpec((2 * H + ETN, H), lambda i: (0, 0)),
            pl.BlockSpec((1, H), lambda i: (0, 0)),
        ],
        out_specs=[
            pl.BlockSpec((r, H), lambda i: (i, 0)),
            pl.BlockSpec((ETN, r, H), lambda i: (0, i, 0)),
            pl.BlockSpec((8, H), lambda i: (0, 0)),
        ],
        out_shape=[
            jax.ShapeDtypeStruct((MCP, H), _F32),
            jax.ShapeDtypeStruct((ETN, MCP, H), _F32),
            jax.ShapeDtypeStruct((8, H), _F32),
        ],
        scratch_shapes=[pltpu.VMEM((8, H), _F32)],
    )(p, wne, bne)


def _final_body(p_ref, sums1_ref, sums2_ref, ws_ref, bs_ref, wsg_ref, bsg_ref,
                gnw_ref, gnb_ref, pf_ref, pgf_ref, acc):
    i = pl.program_id(0)
    n = pl.num_programs(0)

    @pl.when(i == 0)
    def _():
        acc[...] = jnp.zeros_like(acc)

    cf = p_ref[0] + p_ref[1]
    acc[0:1, :] += jnp.sum(cf, axis=0, keepdims=True)

    @pl.when(i == n - 1)
    def _():
        s0 = sums1_ref[0:1, :]
        gsum = sums1_ref[1:2, :]
        ssum = sums1_ref[2:3, :]
        s1 = sums2_ref[0:1, :]
        s2 = acc[0:1, :]
        pre = (jnp.dot(s0, ws_ref[0:H, :], preferred_element_type=_F32)
               + jnp.dot(s1, ws_ref[H:2 * H, :], preferred_element_type=_F32)
               + jnp.dot(s2, ws_ref[2 * H:3 * H, :], preferred_element_type=_F32)
               + bs_ref[...])
        pf_ref[...] = _lrelu(pre)

        pg = _lrelu(gsum)
        sg = _lrelu(ssum)
        y = jnp.dot(pg, wsg_ref[...], preferred_element_type=_F32) + bsg_ref[...]
        r8 = lax.broadcasted_iota(jnp.int32, (H, H), 0) // 8
        c8 = lax.broadcasted_iota(jnp.int32, (H, H), 1) // 8
        m8 = jnp.where(r8 == c8, 1.0 / 8.0, 0.0).astype(_F32)
        m = jnp.dot(y, m8, preferred_element_type=_F32)
        ex2 = jnp.dot(y * y, m8, preferred_element_type=_F32)
        var = ex2 - m * m
        gn = (y - m) * lax.rsqrt(var + 1e-5) * gnw_ref[...] + gnb_ref[...]
        pgf_ref[...] = _lrelu(sg + gn)


def _tc_final(p, sums1, sums2, ws, bs, wsg, bsg, gnw, gnb, r=1024):
    grid = (MCP // r,)
    return pl.pallas_call(
        _final_body,
        grid=grid,
        in_specs=[
            pl.BlockSpec((2, r, H), lambda i: (0, i, 0)),
            pl.BlockSpec((8, H), lambda i: (0, 0)),
            pl.BlockSpec((8, H), lambda i: (0, 0)),
            pl.BlockSpec((3 * H, H), lambda i: (0, 0)),
            pl.BlockSpec((1, H), lambda i: (0, 0)),
            pl.BlockSpec((H, H), lambda i: (0, 0)),
            pl.BlockSpec((1, H), lambda i: (0, 0)),
            pl.BlockSpec((1, H), lambda i: (0, 0)),
            pl.BlockSpec((1, H), lambda i: (0, 0)),
        ],
        out_specs=[
            pl.BlockSpec((1, H), lambda i: (0, 0)),
            pl.BlockSpec((1, H), lambda i: (0, 0)),
        ],
        out_shape=[
            jax.ShapeDtypeStruct((1, H), _F32),
            jax.ShapeDtypeStruct((1, H), _F32),
        ],
        scratch_shapes=[pltpu.VMEM((8, H), _F32)],
    )(p, sums1, sums2, ws, bs, wsg, bsg, gnw, gnb)


_sc_mesh = plsc.VectorSubcoreMesh(core_axis_name="c", subcore_axis_name="s")


@functools.partial(
    pl.kernel,
    mesh=_sc_mesh,
    out_type=jax.ShapeDtypeStruct((NC, MCP, H), _F32),
    compiler_params=pltpu.CompilerParams(use_tc_tiling_on_sc=True),
    scratch_types=(
        [pltpu.VMEM((2, K), jnp.int32) for _ in range(RINGI)]
        + [pltpu.VMEM((K, H), _F32) for _ in range(2 * RING)]
        + [pltpu.SemaphoreType.DMA for _ in range(RINGI + 2 * RING)]
        + [pltpu.VMEM_SHARED((MCP, H), _F32)]
    ),
)
def _sc_edge_pass(a_hbm, c_hbm, idx_hbm, out_hbm,
                  ix0, ix1, ix2, ix3, ra0, ra1, ra2, rc0, rc1, rc2,
                  si0, si1, si2, si3, sg0, sg1, sg2, ss0, ss1, ss2, acc_sh):
    cid = lax.axis_index("c")
    sid = lax.axis_index("s")
    wid = sid * NC + cid
    ix = (ix0, ix1, ix2, ix3)
    ra = (ra0, ra1, ra2)
    rc = (rc0, rc1, rc2)
    si = (si0, si1, si2, si3)
    sg = (sg0, sg1, sg2)
    ss = (ss0, ss1, ss2)

    zv = jnp.zeros((16,), _F32)

    def _zrow(r, carry):
        for c in range(H // 16):
            ra0[r, pl.ds(c * 16, 16)] = zv
        return carry

    lax.fori_loop(0, K, _zrow, 0)
    for z in range(RPT // ZR):
        pltpu.sync_copy(ra0, acc_sh.at[pl.ds(sid * RPT + z * ZR, ZR)])
    plsc.subcore_barrier()

    def _issue_idx(chunk, bi):
        pltpu.async_copy(idx_hbm.at[wid, chunk], ix[bi], si[bi])

    def _wait_idx(chunk, bi):
        pltpu.make_async_copy(idx_hbm.at[wid, chunk], ix[bi], si[bi]).wait()

    def _issue_gather(bi, b):
        pltpu.async_copy(a_hbm.at[ix[bi].at[0]], ra[b], sg[b])
        pltpu.async_copy(c_hbm.at[ix[bi].at[1]], rc[b], sg[b])

    def _wait_gather(bi, b):
        pltpu.make_async_copy(a_hbm.at[ix[bi].at[0]], ra[b], sg[b]).wait()
        pltpu.make_async_copy(c_hbm.at[ix[bi].at[1]], rc[b], sg[b]).wait()

    _issue_idx(0, 0)
    _issue_idx(1, 1)
    _issue_idx(2, 2)
    _wait_idx(0, 0)
    _issue_gather(0, 0)
    _wait_idx(1, 1)
    _issue_gather(1, 1)

    def _group(g, carry):
        for u in range(UNROLL):
            s = g * UNROLL + u
            b = u % RING
            bp = (u + 2) % RING
            bi = u % RINGI
            bip = (u + 2) % RINGI
            bin_ = (u + 3) % RINGI

            @pl.when(jnp.logical_and(s >= 1, s - 1 < NCHUNK))
            def _():
                pltpu.make_async_copy(
                    ra[bp], acc_sh.at[ix[(u + 3) % RINGI].at[0]],
                    ss[bp]).wait()

            @pl.when(s + 3 < NCHUNK)
            def _():
                _issue_idx(s + 3, bin_)

            @pl.when(s + 2 < NCHUNK)
            def _():
                _wait_idx(s + 2, bip)
                _issue_gather(bip, bp)

            @pl.when(s < NCHUNK)
            def _():
                _wait_gather(bi, b)

                def _row(r, rcarry):
                    for c in range(H // 16):
                        x = (ra[b][r, pl.ds(c * 16, 16)]
                             + rc[b][r, pl.ds(c * 16, 16)])
                        ra[b][r, pl.ds(c * 16, 16)] = jnp.maximum(x, x * 0.01)
                    return rcarry

                lax.fori_loop(0, K, _row, 0)
                pltpu.async_copy(ra[b], acc_sh.at[ix[bi].at[0]], ss[b],
                                 add=True)
        return carry

    lax.fori_loop(0, NGROUP, _group, 0)
    plsc.subcore_barrier()

    for z in range(RPT // ZR):
        r0 = sid * RPT + z * ZR
        pltpu.sync_copy(acc_sh.at[pl.ds(r0, ZR)], ra0)
        pltpu.sync_copy(ra0, out_hbm.at[cid, pl.ds(r0, ZR)])


def kernel(child_feats, child_geo_feats, child_exists, edge_type_onehot,
           edge_indices, W_child, b_child, W_second, b_second, W_ne0, b_ne0,
           W_ne1, b_ne1, W_child_geo, b_child_geo, W_second_geo, b_second_geo,
           gn_w, gn_b, W_skip_geo, b_skip_geo):
    child = child_feats[0]
    geo = child_geo_feats[0]
    ex = child_exists[0]
    e_from = edge_indices[0, :, 0].astype(jnp.int32)
    e_to = edge_indices[0, :, 1].astype(jnp.int32)
    t = jnp.argmax(edge_type_onehot[0], axis=1).astype(jnp.int32)
    eidx2a = t * MC + e_to
    eidx2b = t * MCP + e_to
    e_from_w = e_from.reshape(NW, NCHUNK, K)
    idx_a = jnp.stack([e_from_w, eidx2a.reshape(NW, NCHUNK, K)], axis=2)
    idx_b = jnp.stack([e_from_w, eidx2b.reshape(NW, NCHUNK, K)], axis=2)

    a1, c1, sums1 = _tc_prep(
        child, geo, ex, W_child, b_child[None], W_ne0, b_ne0[None],
        W_child_geo, b_child_geo[None], W_skip_geo, b_skip_geo[None])
    p1 = _sc_edge_pass(a1, c1.reshape(ETN * MC, H), idx_a)
    a2, c2, sums2 = _tc_mid(p1, W_ne1, b_ne1[None])
    p2 = _sc_edge_pass(a2, c2.reshape(ETN * MCP, H), idx_b)
    pf, pgf = _tc_final(p2, sums1, sums2, W_second, b_second[None],
                        W_second_geo, b_second_geo[None],
                        gn_w[None], gn_b[None])
    return pf, pgf
